# baseline (speedup 1.0000x reference)
import jax
import jax.numpy as jnp
from jax import lax
from jax.experimental import pallas as pl
from jax.experimental.pallas import tpu as pltpu
from jax.experimental.pallas import tpu_sc as plsc

N = 10000
E = 320000
C_IN = 128
C_V = 64
H = 8

NC = 2
NS = 16
B = 80
EDGES_PER_TILE = E // (NC * NS)
UNROLL = 4
BLOCKS_PER_TILE = 129
IDX_BLOCKS = BLOCKS_PER_TILE + 2
PAD_EPT = IDX_BLOCKS * B
MAIN_ITERS = (BLOCKS_PER_TILE - 1) // UNROLL
ACC_D = 128
N_PAD = 10240
ROWS_PER_TILE = N_PAD // NS
TC_BLK = 1024


def _pre_body(x_ref, wkv_ref, wq_ref, vk_ref, q_ref):
    x = x_ref[...]
    vk_ref[...] = jnp.dot(x, wkv_ref[...], preferred_element_type=jnp.float32)
    q_ref[...] = jnp.dot(x, wq_ref[...], preferred_element_type=jnp.float32)


def _post_body(x_ref, a0_ref, a1_ref, wpt_ref, wpb_ref, r_ref, out_ref):
    a = a0_ref[...] + a1_ref[...]
    w = a[:, :C_V]
    den = a[:, C_V:C_V + H]
    den_rep = jnp.dot(den, r_ref[...], preferred_element_type=jnp.float32)
    feat = w / (den_rep + 1e-9)
    out_ref[...] = (
        jnp.dot(x_ref[...], wpt_ref[...], preferred_element_type=jnp.float32)
        + jnp.dot(feat, wpb_ref[...], preferred_element_type=jnp.float32))


def _sc_body(vk_hbm, q_hbm, src_hbm, dst_hbm, zeros_hbm, out_hbm,
             sidx2, didx2, vk0, vk1, q0, q1, acc_sh,
             semi0, semi1, semi2, semi3,
             semgv0, semgv1, semgq0, semgq1, semc0, semc1):
    cid = lax.axis_index("c")
    sid = lax.axis_index("s")
    zeros16 = jnp.zeros((16,), jnp.float32)
    vks = (vk0, vk1)
    qs = (q0, q1)
    semi = (semi0, semi1, semi2, semi3)
    semgv = (semgv0, semgv1)
    semgq = (semgq0, semgq1)
    semc = (semc0, semc1)

    row0 = sid * ROWS_PER_TILE
    pltpu.sync_copy(zeros_hbm.at[pl.ds(row0, ROWS_PER_TILE)],
                    acc_sh.at[pl.ds(row0, ROWS_PER_TILE)])

    plsc.subcore_barrier()

    tile_edge_base = (cid * NS + sid) * PAD_EPT

    def issue_idx(j, s):
        base = tile_edge_base + j * B
        pltpu.async_copy(src_hbm.at[pl.ds(base, B)], sidx2.at[s], semi[s])
        pltpu.async_copy(dst_hbm.at[pl.ds(base, B)], didx2.at[s], semi[s])

    def wait_idx(s):
        pltpu.make_async_copy(src_hbm.at[pl.ds(0, B)], sidx2.at[s], semi[s]).wait()
        pltpu.make_async_copy(dst_hbm.at[pl.ds(0, B)], didx2.at[s], semi[s]).wait()

    def issue_gather(s6, sb):
        pltpu.async_copy(vk_hbm.at[sidx2.at[s6]], vks[sb], semgv[sb])
        pltpu.async_copy(q_hbm.at[didx2.at[s6]], qs[sb], semgq[sb])

    def wait_gather(s6, sb):
        pltpu.make_async_copy(vk_hbm.at[sidx2.at[s6]], vks[sb], semgv[sb]).wait()
        pltpu.make_async_copy(q_hbm.at[didx2.at[s6]], qs[sb], semgq[sb]).wait()

    def issue_scatter(s6, sb):
        pltpu.async_copy(vks[sb], acc_sh.at[didx2.at[s6]], semc[sb], add=True)

    def wait_scatter(s6, sb):
        pltpu.make_async_copy(vks[sb], acc_sh.at[didx2.at[s6]], semc[sb]).wait()

    def compute(vkb, qb):
        def group_body(g, _):
            eoff = lax.iota(jnp.int32, 16) + g * 16
            dots = [zeros16] * H
            col = jnp.zeros((16,), jnp.int32)
            for c in range(C_V):
                kc = plsc.load_gather(vkb, [eoff, col + C_V])
                qc = plsc.load_gather(qb, [eoff, col])
                dots[c // 8] = dots[c // 8] + kc * qc
                col = col + 1
            ps = []
            for h in range(H):
                p = jnp.exp(dots[h] * 0.125)
                ps.append(p)
                plsc.store_scatter(vkb, [eoff, col], p)
                col = col + 1
            col = jnp.zeros((16,), jnp.int32)
            for c in range(C_V):
                vc = plsc.load_gather(vkb, [eoff, col])
                plsc.store_scatter(vkb, [eoff, col], vc * ps[c // 8])
                col = col + 1
            return 0

        lax.fori_loop(0, B // 16, group_body, 0)

    def step(i, b, first=False):
        if not first:
            wait_scatter((b - 1) % UNROLL, (b - 1) % 2)
        wait_gather(b, b % 2)
        issue_idx(i + 2, (b + 2) % UNROLL)
        wait_idx((b + 1) % UNROLL)
        issue_gather((b + 1) % UNROLL, (b + 1) % 2)
        compute(vks[b % 2], qs[b % 2])
        issue_scatter(b, b % 2)

    issue_idx(0, 0)
    issue_idx(1, 1)
    wait_idx(0)
    issue_gather(0, 0)
    step(0, 0, first=True)

    def quad(t, _):
        for bp in range(UNROLL):
            i = 1 + t * UNROLL + bp
            step(i, (1 + bp) % UNROLL)
        return 0
    lax.fori_loop(0, MAIN_ITERS, quad, 0)

    wait_scatter(0, 0)
    wait_gather(1, 1)
    wait_idx(2)

    plsc.subcore_barrier()
    pltpu.sync_copy(acc_sh.at[pl.ds(row0, ROWS_PER_TILE)],
                    out_hbm.at[pl.ds(cid * N_PAD + row0, ROWS_PER_TILE)])


def kernel(node_feats_0, edge_index, W_kv, W_q, W_proj):
    x = node_feats_0[:, :, 0]
    x_pad = jnp.pad(x, ((0, N_PAD - N), (0, 0)))

    vk, q = pl.pallas_call(
        _pre_body,
        grid=(N_PAD // TC_BLK,),
        in_specs=[
            pl.BlockSpec((TC_BLK, C_IN), lambda i: (i, 0)),
            pl.BlockSpec((C_IN, 2 * C_V), lambda i: (0, 0)),
            pl.BlockSpec((C_IN, 2 * C_V), lambda i: (0, 0)),
        ],
        out_specs=[
            pl.BlockSpec((TC_BLK, 2 * C_V), lambda i: (i, 0)),
            pl.BlockSpec((TC_BLK, 2 * C_V), lambda i: (i, 0)),
        ],
        out_shape=[
            jax.ShapeDtypeStruct((N_PAD, 2 * C_V), jnp.float32),
            jax.ShapeDtypeStruct((N_PAD, 2 * C_V), jnp.float32),
        ],
    )(x_pad, W_kv, jnp.pad(W_q, ((0, 0), (0, C_V))))

    pad_cols = PAD_EPT - EDGES_PER_TILE
    src = jnp.pad(edge_index[0].reshape(NC * NS, EDGES_PER_TILE),
                  ((0, 0), (0, pad_cols))).reshape(-1)
    dst = jnp.pad(edge_index[1].reshape(NC * NS, EDGES_PER_TILE),
                  ((0, 0), (0, pad_cols)),
                  constant_values=N_PAD - 1).reshape(-1)
    zeros = jnp.zeros((N_PAD, ACC_D), jnp.float32)

    sc_edges = pl.kernel(
        _sc_body,
        out_type=jax.ShapeDtypeStruct((NC * N_PAD, ACC_D), jnp.float32),
        mesh=plsc.VectorSubcoreMesh(core_axis_name="c", subcore_axis_name="s"),
        compiler_params=pltpu.CompilerParams(needs_layout_passes=False),
        scratch_types=[
            pltpu.VMEM((UNROLL, B), jnp.int32),
            pltpu.VMEM((UNROLL, B), jnp.int32),
            pltpu.VMEM((B, 2 * C_V), jnp.float32),
            pltpu.VMEM((B, 2 * C_V), jnp.float32),
            pltpu.VMEM((B, 2 * C_V), jnp.float32),
            pltpu.VMEM((B, 2 * C_V), jnp.float32),
            pltpu.VMEM_SHARED((N_PAD, ACC_D), jnp.float32),
        ] + [pltpu.SemaphoreType.DMA] * 10,
    )
    acc = sc_edges(vk, q, src, dst, zeros)

    wp_top = W_proj[:C_IN]
    wp_bot = W_proj[C_IN:]
    r_mat = jnp.kron(jnp.eye(H, dtype=jnp.float32),
                     jnp.ones((1, H), dtype=jnp.float32))

    post_blk = 1000
    out2d = pl.pallas_call(
        _post_body,
        grid=(N // post_blk,),
        in_specs=[
            pl.BlockSpec((post_blk, C_IN), lambda i: (i, 0)),
            pl.BlockSpec((post_blk, ACC_D), lambda i: (i, 0)),
            pl.BlockSpec((post_blk, ACC_D), lambda i: (i, 0)),
            pl.BlockSpec((C_IN, C_IN), lambda i: (0, 0)),
            pl.BlockSpec((C_V, C_IN), lambda i: (0, 0)),
            pl.BlockSpec((H, C_V), lambda i: (0, 0)),
        ],
        out_specs=pl.BlockSpec((post_blk, C_IN), lambda i: (i, 0)),
        out_shape=jax.ShapeDtypeStruct((N, C_IN), jnp.float32),
    )(x, acc[:N], acc[N_PAD:N_PAD + N], wp_top, wp_bot, r_mat)

    return out2d[:, :, None]


# speedup vs baseline: 1.0000x; 1.0000x over previous
import jax
import jax.numpy as jnp
from jax import lax
from jax.experimental import pallas as pl
from jax.experimental.pallas import tpu as pltpu
from jax.experimental.pallas import tpu_sc as plsc

N = 10000
E = 320000
C_IN = 128
C_V = 64
H = 8

NC = 2
NS = 16
B = 80
EDGES_PER_TILE = E // (NC * NS)
UNROLL = 4
BLOCKS_PER_TILE = 129
IDX_BLOCKS = BLOCKS_PER_TILE + 2
PAD_EPT = IDX_BLOCKS * B
MAIN_ITERS = (BLOCKS_PER_TILE - 1) // UNROLL
ACC_D = 128
N_PAD = 10240
ROWS_PER_TILE = N_PAD // NS
TC_BLK = 1024


def _pre_body(x_ref, wkv_ref, wq_ref, vk_ref, q_ref):
    x = x_ref[...]
    vk_ref[...] = jnp.dot(x, wkv_ref[...], preferred_element_type=jnp.float32)
    q_ref[...] = jnp.dot(x, wq_ref[...], preferred_element_type=jnp.float32)


def _post_body(x_ref, a0_ref, a1_ref, wpt_ref, wpb_ref, r_ref, out_ref):
    a = a0_ref[...] + a1_ref[...]
    w = a[:, :C_V]
    den = a[:, C_V:C_V + H]
    den_rep = jnp.dot(den, r_ref[...], preferred_element_type=jnp.float32)
    feat = w / (den_rep + 1e-9)
    out_ref[...] = (
        jnp.dot(x_ref[...], wpt_ref[...], preferred_element_type=jnp.float32)
        + jnp.dot(feat, wpb_ref[...], preferred_element_type=jnp.float32))


def _sc_body(vk_hbm, q_hbm, src_hbm, dst_hbm, zeros_hbm, out_hbm,
             sidx2, didx2, vk0, vk1, q0, q1, acc_sh,
             semi0, semi1, semi2, semi3,
             semgv0, semgv1, semgq0, semgq1, semc0, semc1):
    cid = lax.axis_index("c")
    sid = lax.axis_index("s")
    zeros16 = jnp.zeros((16,), jnp.float32)
    vks = (vk0, vk1)
    qs = (q0, q1)
    semi = (semi0, semi1, semi2, semi3)
    semgv = (semgv0, semgv1)
    semgq = (semgq0, semgq1)
    semc = (semc0, semc1)

    row0 = sid * ROWS_PER_TILE
    pltpu.sync_copy(zeros_hbm.at[pl.ds(row0, ROWS_PER_TILE)],
                    acc_sh.at[pl.ds(row0, ROWS_PER_TILE)])

    plsc.subcore_barrier()

    tile_edge_base = (cid * NS + sid) * PAD_EPT

    def issue_idx(j, s):
        base = tile_edge_base + j * B
        pltpu.async_copy(src_hbm.at[pl.ds(base, B)], sidx2.at[s], semi[s])
        pltpu.async_copy(dst_hbm.at[pl.ds(base, B)], didx2.at[s], semi[s])

    def wait_idx(s):
        pltpu.make_async_copy(src_hbm.at[pl.ds(0, B)], sidx2.at[s], semi[s]).wait()
        pltpu.make_async_copy(dst_hbm.at[pl.ds(0, B)], didx2.at[s], semi[s]).wait()

    def issue_gather(s6, sb):
        pltpu.async_copy(vk_hbm.at[sidx2.at[s6]], vks[sb], semgv[sb])
        pltpu.async_copy(q_hbm.at[didx2.at[s6]], qs[sb], semgq[sb])

    def wait_gather(s6, sb):
        pltpu.make_async_copy(vk_hbm.at[sidx2.at[s6]], vks[sb], semgv[sb]).wait()
        pltpu.make_async_copy(q_hbm.at[didx2.at[s6]], qs[sb], semgq[sb]).wait()

    def issue_scatter(s6, sb):
        pltpu.async_copy(qs[sb], acc_sh.at[didx2.at[s6]], semc[sb], add=True)

    def wait_scatter(s6, sb):
        pltpu.make_async_copy(qs[sb], acc_sh.at[didx2.at[s6]], semc[sb]).wait()

    def compute(vkb, qb):
        def group_body(g, _):
            eoff = lax.iota(jnp.int32, 16) + g * 16
            dots = [zeros16] * H
            col = jnp.zeros((16,), jnp.int32)
            for c in range(C_V):
                kc = plsc.load_gather(vkb, [eoff, col + C_V])
                qc = plsc.load_gather(qb, [eoff, col])
                dots[c // 8] = dots[c // 8] + kc * qc
                col = col + 1
            ps = []
            for h in range(H):
                p = jnp.exp(dots[h] * 0.125)
                ps.append(p)
                plsc.store_scatter(qb, [eoff, col], p)
                col = col + 1
            col = jnp.zeros((16,), jnp.int32)
            for c in range(C_V):
                vc = plsc.load_gather(vkb, [eoff, col])
                plsc.store_scatter(qb, [eoff, col], vc * ps[c // 8])
                col = col + 1
            return 0

        lax.fori_loop(0, B // 16, group_body, 0)

    def step(i, b, first=False):
        if not first:
            wait_scatter((b - 1) % UNROLL, (b - 1) % 2)
        wait_gather(b, b % 2)
        issue_idx(i + 2, (b + 2) % UNROLL)
        wait_idx((b + 1) % UNROLL)
        issue_gather((b + 1) % UNROLL, (b + 1) % 2)
        compute(vks[b % 2], qs[b % 2])
        issue_scatter(b, b % 2)

    issue_idx(0, 0)
    issue_idx(1, 1)
    wait_idx(0)
    issue_gather(0, 0)
    step(0, 0, first=True)

    def quad(t, _):
        for bp in range(UNROLL):
            i = 1 + t * UNROLL + bp
            step(i, (1 + bp) % UNROLL)
        return 0
    lax.fori_loop(0, MAIN_ITERS, quad, 0)

    wait_scatter(0, 0)
    wait_gather(1, 1)
    wait_idx(2)

    plsc.subcore_barrier()
    pltpu.sync_copy(acc_sh.at[pl.ds(row0, ROWS_PER_TILE)],
                    out_hbm.at[pl.ds(cid * N_PAD + row0, ROWS_PER_TILE)])


def kernel(node_feats_0, edge_index, W_kv, W_q, W_proj):
    x = node_feats_0[:, :, 0]
    x_pad = jnp.pad(x, ((0, N_PAD - N), (0, 0)))

    vk, q = pl.pallas_call(
        _pre_body,
        grid=(N_PAD // TC_BLK,),
        in_specs=[
            pl.BlockSpec((TC_BLK, C_IN), lambda i: (i, 0)),
            pl.BlockSpec((C_IN, 2 * C_V), lambda i: (0, 0)),
            pl.BlockSpec((C_IN, 2 * C_V), lambda i: (0, 0)),
        ],
        out_specs=[
            pl.BlockSpec((TC_BLK, 2 * C_V), lambda i: (i, 0)),
            pl.BlockSpec((TC_BLK, 2 * C_V), lambda i: (i, 0)),
        ],
        out_shape=[
            jax.ShapeDtypeStruct((N_PAD, 2 * C_V), jnp.float32),
            jax.ShapeDtypeStruct((N_PAD, 2 * C_V), jnp.float32),
        ],
    )(x_pad, W_kv, jnp.pad(W_q, ((0, 0), (0, C_V))))

    pad_cols = PAD_EPT - EDGES_PER_TILE
    src = jnp.pad(edge_index[0].reshape(NC * NS, EDGES_PER_TILE),
                  ((0, 0), (0, pad_cols))).reshape(-1)
    dst = jnp.pad(edge_index[1].reshape(NC * NS, EDGES_PER_TILE),
                  ((0, 0), (0, pad_cols)),
                  constant_values=N_PAD - 1).reshape(-1)
    zeros = jnp.zeros((N_PAD, ACC_D), jnp.float32)

    sc_edges = pl.kernel(
        _sc_body,
        out_type=jax.ShapeDtypeStruct((NC * N_PAD, ACC_D), jnp.float32),
        mesh=plsc.VectorSubcoreMesh(core_axis_name="c", subcore_axis_name="s"),
        compiler_params=pltpu.CompilerParams(needs_layout_passes=False),
        scratch_types=[
            pltpu.VMEM((UNROLL, B), jnp.int32),
            pltpu.VMEM((UNROLL, B), jnp.int32),
            pltpu.VMEM((B, 2 * C_V), jnp.float32),
            pltpu.VMEM((B, 2 * C_V), jnp.float32),
            pltpu.VMEM((B, 2 * C_V), jnp.float32),
            pltpu.VMEM((B, 2 * C_V), jnp.float32),
            pltpu.VMEM_SHARED((N_PAD, ACC_D), jnp.float32),
        ] + [pltpu.SemaphoreType.DMA] * 10,
    )
    acc = sc_edges(vk, q, src, dst, zeros)

    wp_top = W_proj[:C_IN]
    wp_bot = W_proj[C_IN:]
    r_mat = jnp.kron(jnp.eye(H, dtype=jnp.float32),
                     jnp.ones((1, H), dtype=jnp.float32))

    post_blk = 1000
    out2d = pl.pallas_call(
        _post_body,
        grid=(N // post_blk,),
        in_specs=[
            pl.BlockSpec((post_blk, C_IN), lambda i: (i, 0)),
            pl.BlockSpec((post_blk, ACC_D), lambda i: (i, 0)),
            pl.BlockSpec((post_blk, ACC_D), lambda i: (i, 0)),
            pl.BlockSpec((C_IN, C_IN), lambda i: (0, 0)),
            pl.BlockSpec((C_V, C_IN), lambda i: (0, 0)),
            pl.BlockSpec((H, C_V), lambda i: (0, 0)),
        ],
        out_specs=pl.BlockSpec((post_blk, C_IN), lambda i: (i, 0)),
        out_shape=jax.ShapeDtypeStruct((N, C_IN), jnp.float32),
    )(x, acc[:N], acc[N_PAD:N_PAD + N], wp_top, wp_bot, r_mat)

    return out2d[:, :, None]
